# baseline (speedup 1.0000x reference)
import functools

import jax
import jax.numpy as jnp
from jax import lax
from jax.experimental import pallas as pl
from jax.experimental.pallas import tpu as pltpu
from jax.experimental.pallas import tpu_sc as plsc

B = 128
N = 8192
L = 16
NC, NS = 2, 16
NW = NC * NS
RPW = B // NW

LO0 = 0x35000000
HI0 = 0x3F800000
SHIFT = 19
NB = (HI0 - LO0) >> SHIFT
NBV = NB // L


def _sc_body(scores_hbm, seqlen_hbm, out_hbm, row_v, buf_v, hist_v, seq_v, vl_v):
    wid = lax.axis_index("s") * NC + lax.axis_index("c")
    pltpu.sync_copy(seqlen_hbm, seq_v.at[pl.ds(0, B)])
    lanes = lax.iota(jnp.int32, L)
    zeros_f = jnp.zeros((L,), jnp.float32)
    zeros_i = jnp.zeros((L,), jnp.int32)
    ones_i = jnp.ones((L,), jnp.int32)
    neg1_i = jnp.full((L,), -1, jnp.int32)
    lo0_vec = jnp.full((L,), LO0, jnp.int32)

    def row_body(i, vl_vec):
        row = wid * RPW + i
        pltpu.sync_copy(scores_hbm.at[row], row_v)
        s = seq_v[pl.ds(row, L)][0]
        s_vec = jnp.full((L,), s, jnp.int32)
        k = (s >> 4) + 1
        k_vec = jnp.full((L,), k, jnp.int32)
        nchunks = (s + (L - 1)) >> 4

        def mask_body(jc, _):
            pos = lanes + jc * L
            d = row_v[pl.ds(jc * L, L)]
            row_v[pl.ds(jc * L, L)] = jnp.where(pos < s_vec, d, zeros_f)
            return 0
        lax.fori_loop(s >> 4, nchunks, mask_body, 0)

        for j in range(NBV):
            hist_v[pl.ds(j * L, L)] = zeros_i

        def hist_body(jc, _):
            d = row_v[pl.ds(jc * L, L)]
            kb = (plsc.bitcast(d, jnp.int32) - lo0_vec) >> SHIFT
            plsc.addupdate_scatter(hist_v, [kb], ones_i, mask=d > zeros_f)
            return 0
        lax.fori_loop(0, nchunks, hist_body, 0)

        carry = zeros_i
        bstar_acc = neg1_i
        for j in range(NBV - 1, -1, -1):
            v = hist_v[pl.ds(j * L, L)]
            c = plsc.cumsum(lax.rev(v, (0,)))
            suffix = lax.rev(c, (0,))
            g = carry + suffix
            idx_vec = lanes + j * L
            bstar_acc = jnp.maximum(bstar_acc, jnp.where(g >= k_vec, idx_vec, neg1_i))
            carry = carry + jnp.full((L,), c[L - 1], jnp.int32)
        bstar = jnp.max(bstar_acc)
        bstar_vec = jnp.full((L,), bstar, jnp.int32)

        def cmp_body(jc, st):
            sacc, cacc, off = st
            d = row_v[pl.ds(jc * L, L)]
            kb = (plsc.bitcast(d, jnp.int32) - lo0_vec) >> SHIFT
            above = kb > bstar_vec
            sacc = sacc + jnp.where(above, d, zeros_f)
            cacc = cacc + jnp.where(above, ones_i, zeros_i)
            m_eq = kb == bstar_vec
            plsc.store_compressed(buf_v.at[pl.ds(off, L)], d, mask=m_eq)
            pc = plsc.all_reduce_population_count(m_eq)
            return sacc, cacc, off + pc[0]
        sacc, cacc, nb = lax.fori_loop(
            0, nchunks, cmp_body, (zeros_f, zeros_i, jnp.int32(0)))
        sum_above = jnp.sum(sacc)
        kp = k - jnp.sum(cacc)
        kp_vec = jnp.full((L,), kp, jnp.int32)
        nb_vec = jnp.full((L,), nb, jnp.int32)

        jc = nb >> 4
        pos = lanes + jc * L
        dd = buf_v[pl.ds(jc * L, L)]
        buf_v[pl.ds(jc * L, L)] = jnp.where(pos < nb_vec, dd, zeros_f)
        nchunks_b = (nb + (L - 1)) >> 4

        lo_b = jnp.int32(LO0) + (bstar << SHIFT)

        def search_body(it, st):
            lo, hi = st
            mid = (lo + hi) >> 1
            t_vec = plsc.bitcast(jnp.full((L,), mid, jnp.int32), jnp.float32)

            def cnt_body(jb, acc):
                d = buf_v[pl.ds(jb * L, L)]
                return acc + jnp.where(d >= t_vec, ones_i, zeros_i)
            cnt = jnp.sum(lax.fori_loop(0, nchunks_b, cnt_body, zeros_i))
            ge = cnt >= kp
            return jnp.where(ge, mid, lo), jnp.where(ge, hi, mid)
        lo, hi = lax.fori_loop(0, SHIFT, search_body, (lo_b, lo_b + (1 << SHIFT)))
        t_vec = plsc.bitcast(jnp.full((L,), lo, jnp.int32), jnp.float32)

        def fin_body(jb, st):
            sa, ca = st
            d = buf_v[pl.ds(jb * L, L)]
            gt = d > t_vec
            return sa + jnp.where(gt, d, zeros_f), ca + jnp.where(gt, ones_i, zeros_i)
        sfin, cfin = lax.fori_loop(0, nchunks_b, fin_body, (zeros_f, zeros_i))

        tot_vec = (jnp.full((L,), sum_above) + jnp.full((L,), jnp.sum(sfin))
                   + (kp_vec - jnp.full((L,), jnp.sum(cfin), jnp.int32)
                      ).astype(jnp.float32) * t_vec)
        i_vec = jnp.full((L,), i, jnp.int32)
        vl_vec = jnp.where(lanes == i_vec, tot_vec, vl_vec)
        vl_vec = jnp.where(lanes == i_vec + RPW, k_vec.astype(jnp.float32), vl_vec)
        return vl_vec

    vl_v[...] = lax.fori_loop(0, RPW, row_body, zeros_f)
    pltpu.sync_copy(vl_v, out_hbm.at[wid])


_sc_topk = pl.kernel(
    _sc_body,
    out_type=jax.ShapeDtypeStruct((NW, L), jnp.float32),
    mesh=plsc.VectorSubcoreMesh(core_axis_name="c", subcore_axis_name="s"),
    scratch_types=[
        pltpu.VMEM((N,), jnp.float32),
        pltpu.VMEM((N,), jnp.float32),
        pltpu.VMEM((NB,), jnp.int32),
        pltpu.VMEM((B + L,), jnp.int32),
        pltpu.VMEM((L,), jnp.float32),
    ],
    compiler_params=pltpu.CompilerParams(needs_layout_passes=False),
)


def _tc_bce_body(vl_ref, lab_ref, out_ref):
    raw = vl_ref[...]
    v = raw[:, :RPW] / raw[:, RPW:2 * RPW]
    lab = lab_ref[...]
    terms = lab * jnp.log(v) + (1.0 - lab) * jnp.log(1.0 - v)
    out_ref[0, 0] = -jnp.sum(terms) / B


_tc_bce = pl.pallas_call(
    _tc_bce_body,
    out_shape=jax.ShapeDtypeStruct((1, 1), jnp.float32),
    out_specs=pl.BlockSpec(memory_space=pltpu.SMEM),
)


@jax.jit
def kernel(scores, label, seqlen):
    vl_raw = _sc_topk(scores, seqlen)
    loss = _tc_bce(vl_raw, label.reshape(NW, RPW))
    return loss[0, 0]


# speedup vs baseline: 1.0023x; 1.0023x over previous
import functools

import jax
import jax.numpy as jnp
from jax import lax
from jax.experimental import pallas as pl
from jax.experimental.pallas import tpu as pltpu
from jax.experimental.pallas import tpu_sc as plsc

B = 128
N = 8192
L = 16
NC, NS = 2, 16
NW = NC * NS
RPW = B // NW

LO0 = 0x35000000
HI0 = 0x3F800000
SHIFT = 19
NB = (HI0 - LO0) >> SHIFT
NBV = NB // L


def _sc_body(scores_hbm, seqlen_hbm, out_hbm, row_v, buf_v, hist_v, seq_v, vl_v):
    wid = lax.axis_index("s") * NC + lax.axis_index("c")
    pltpu.sync_copy(seqlen_hbm, seq_v.at[pl.ds(0, B)])
    lanes = lax.iota(jnp.int32, L)
    zeros_f = jnp.zeros((L,), jnp.float32)
    zeros_i = jnp.zeros((L,), jnp.int32)
    ones_i = jnp.ones((L,), jnp.int32)
    neg1_i = jnp.full((L,), -1, jnp.int32)
    lo0_vec = jnp.full((L,), LO0, jnp.int32)

    def row_body(i, vl_vec):
        row = wid * RPW + i
        pltpu.sync_copy(scores_hbm.at[row], row_v)
        s = seq_v[pl.ds(row, L)][0]
        s_vec = jnp.full((L,), s, jnp.int32)
        k = (s >> 4) + 1
        k_vec = jnp.full((L,), k, jnp.int32)
        nblk = (s + 127) >> 7

        def mask_body(jc, _):
            pos = lanes + jc * L
            d = row_v[pl.ds(jc * L, L)]
            row_v[pl.ds(jc * L, L)] = jnp.where(pos < s_vec, d, zeros_f)
            return 0
        lax.fori_loop(s >> 4, nblk * 8, mask_body, 0)

        for j in range(NBV):
            hist_v[pl.ds(j * L, L)] = zeros_i

        def hist_body(jb, _):
            base = jb * (8 * L)
            for u in range(8):
                d = row_v[pl.ds(base + u * L, L)]
                kb = (plsc.bitcast(d, jnp.int32) - lo0_vec) >> SHIFT
                plsc.addupdate_scatter(hist_v, [kb], ones_i, mask=d > zeros_f)
            return 0
        lax.fori_loop(0, nblk, hist_body, 0)

        carry = zeros_i
        bstar_acc = neg1_i
        for j in range(NBV - 1, -1, -1):
            v = hist_v[pl.ds(j * L, L)]
            c = plsc.cumsum(lax.rev(v, (0,)))
            suffix = lax.rev(c, (0,))
            g = carry + suffix
            idx_vec = lanes + j * L
            bstar_acc = jnp.maximum(bstar_acc, jnp.where(g >= k_vec, idx_vec, neg1_i))
            carry = carry + jnp.full((L,), c[L - 1], jnp.int32)
        bstar = jnp.max(bstar_acc)
        bstar_vec = jnp.full((L,), bstar, jnp.int32)

        cacc = zeros_i
        for j in range(NBV):
            v = hist_v[pl.ds(j * L, L)]
            idx_vec = lanes + j * L
            cacc = cacc + jnp.where(idx_vec > bstar_vec, v, zeros_i)
        kp = k - jnp.sum(cacc)

        def cmp_body(jb, st):
            sacc, off = st
            base = jb * (8 * L)
            for u in range(8):
                d = row_v[pl.ds(base + u * L, L)]
                kb = (plsc.bitcast(d, jnp.int32) - lo0_vec) >> SHIFT
                sacc = sacc + jnp.where(kb > bstar_vec, d, zeros_f)
                m_eq = kb == bstar_vec
                plsc.store_compressed(buf_v.at[pl.ds(off, L)], d, mask=m_eq)
                pc = plsc.all_reduce_population_count(m_eq)
                off = off + pc[0]
            return sacc, off
        sacc, nb = lax.fori_loop(
            0, nblk, cmp_body, (zeros_f, jnp.int32(0)))
        sum_above = jnp.sum(sacc)
        kp_vec = jnp.full((L,), kp, jnp.int32)
        nb_vec = jnp.full((L,), nb, jnp.int32)

        jc = nb >> 4
        pos = lanes + jc * L
        dd = buf_v[pl.ds(jc * L, L)]
        buf_v[pl.ds(jc * L, L)] = jnp.where(pos < nb_vec, dd, zeros_f)
        nchunks_b = (nb + (L - 1)) >> 4

        lo_b = jnp.int32(LO0) + (bstar << SHIFT)

        def search_body(it, st):
            lo, hi = st
            mid = (lo + hi) >> 1
            t_vec = plsc.bitcast(jnp.full((L,), mid, jnp.int32), jnp.float32)

            def cnt_body(jb, acc):
                d = buf_v[pl.ds(jb * L, L)]
                return acc + jnp.where(d >= t_vec, ones_i, zeros_i)
            cnt = jnp.sum(lax.fori_loop(0, nchunks_b, cnt_body, zeros_i))
            ge = cnt >= kp
            return jnp.where(ge, mid, lo), jnp.where(ge, hi, mid)
        lo, hi = lax.fori_loop(0, SHIFT, search_body, (lo_b, lo_b + (1 << SHIFT)))
        t_vec = plsc.bitcast(jnp.full((L,), lo, jnp.int32), jnp.float32)

        def fin_body(jb, st):
            sa, ca = st
            d = buf_v[pl.ds(jb * L, L)]
            gt = d > t_vec
            return sa + jnp.where(gt, d, zeros_f), ca + jnp.where(gt, ones_i, zeros_i)
        sfin, cfin = lax.fori_loop(0, nchunks_b, fin_body, (zeros_f, zeros_i))

        tot_vec = (jnp.full((L,), sum_above) + jnp.full((L,), jnp.sum(sfin))
                   + (kp_vec - jnp.full((L,), jnp.sum(cfin), jnp.int32)
                      ).astype(jnp.float32) * t_vec)
        i_vec = jnp.full((L,), i, jnp.int32)
        vl_vec = jnp.where(lanes == i_vec, tot_vec, vl_vec)
        vl_vec = jnp.where(lanes == i_vec + RPW, k_vec.astype(jnp.float32), vl_vec)
        return vl_vec

    vl_v[...] = lax.fori_loop(0, RPW, row_body, zeros_f)
    pltpu.sync_copy(vl_v, out_hbm.at[wid])


_sc_topk = pl.kernel(
    _sc_body,
    out_type=jax.ShapeDtypeStruct((NW, L), jnp.float32),
    mesh=plsc.VectorSubcoreMesh(core_axis_name="c", subcore_axis_name="s"),
    scratch_types=[
        pltpu.VMEM((N,), jnp.float32),
        pltpu.VMEM((N,), jnp.float32),
        pltpu.VMEM((NB,), jnp.int32),
        pltpu.VMEM((B + L,), jnp.int32),
        pltpu.VMEM((L,), jnp.float32),
    ],
    compiler_params=pltpu.CompilerParams(needs_layout_passes=False),
)


def _tc_bce_body(vl_ref, lab_ref, out_ref):
    raw = vl_ref[...]
    v = raw[:, :RPW] / raw[:, RPW:2 * RPW]
    lab = lab_ref[...]
    terms = lab * jnp.log(v) + (1.0 - lab) * jnp.log(1.0 - v)
    out_ref[0, 0] = -jnp.sum(terms) / B


_tc_bce = pl.pallas_call(
    _tc_bce_body,
    out_shape=jax.ShapeDtypeStruct((1, 1), jnp.float32),
    out_specs=pl.BlockSpec(memory_space=pltpu.SMEM),
)


@jax.jit
def kernel(scores, label, seqlen):
    vl_raw = _sc_topk(scores, seqlen)
    loss = _tc_bce(vl_raw, label.reshape(NW, RPW))
    return loss[0, 0]
